# Initial kernel scaffold; baseline (speedup 1.0000x reference)
import jax
import jax.numpy as jnp
from jax.experimental import pallas as pl


def kernel(inputs, probs, actions, inserts):
    raise NotImplementedError("write your pallas kernel here")


# speedup vs baseline: 209.0021x; 209.0021x over previous
import functools

import jax
import jax.numpy as jnp
from jax import lax
from jax.experimental import pallas as pl
from jax.experimental.pallas import tpu as pltpu
from jax.experimental.pallas import tpu_sc as plsc

_NC, _NS = 2, 16
_NW = _NC * _NS
_L = 16


def _sc_scatter_tables(inserts, last_action):
    b, s = inserts.shape
    rpw = b // _NW
    chunk = 64
    nch = rpw // chunk
    skip = s // chunk
    mesh = plsc.VectorSubcoreMesh(
        core_axis_name="c", subcore_axis_name="s", num_cores=_NC, num_subcores=_NS
    )

    @functools.partial(
        pl.kernel,
        out_type=jax.ShapeDtypeStruct((_NW, s, s), jnp.int32),
        mesh=mesh,
        scratch_types=[
            pltpu.VMEM((chunk, s), jnp.int32),
            pltpu.VMEM((rpw,), jnp.int32),
            pltpu.VMEM((s, s), jnp.int32),
        ],
        compiler_params=pltpu.CompilerParams(needs_layout_passes=False),
    )
    def sc_kernel(ins_hbm, la_hbm, out_hbm, ins_buf, la_buf, table):
        cid = lax.axis_index("c")
        sid = lax.axis_index("s")
        wid = sid * _NC + cid
        rbase = wid * rpw
        lanes = lax.iota(jnp.int32, _L)
        neg1 = jnp.full((_L,), -1, jnp.int32)

        def init_body(t, _):
            for jb in range(s // _L):
                table[t, pl.ds(jb * _L, _L)] = neg1
            return 0

        lax.fori_loop(0, s, init_body, 0)

        pltpu.sync_copy(la_hbm.at[pl.ds(rbase, rpw)], la_buf)
        start = jnp.where(wid == 0, skip, 0)

        def chunk_body(ci, _):
            row0 = rbase + ci * chunk
            pltpu.sync_copy(ins_hbm.at[pl.ds(row0, chunk)], ins_buf)

            def grp_body(g, _):
                lavec = la_buf[pl.ds(ci * chunk + g * _L, _L)]
                for q in range(_L):
                    rr = g * _L + q
                    rglob = row0 + rr
                    code = jnp.full((_L,), rglob * s + lavec[q], jnp.int32)
                    for jb in range(s // _L):
                        cvals = ins_buf[rr, pl.ds(jb * _L, _L)]
                        jvals = lanes + jb * _L
                        plsc.store_scatter(table, [jvals, cvals], code)
                return 0

            lax.fori_loop(0, chunk // _L, grp_body, 0)
            return 0

        lax.fori_loop(start, nch, chunk_body, 0)
        pltpu.sync_copy(table, out_hbm.at[wid])

    return sc_kernel(inserts, last_action)


def _tc_body(bt_ref, insT_ref, codes_ref, x_ref, y_ref, out_ref, r_ref,
             bigcode_ref, rank_ref):
    s = out_ref.shape[0]
    f32 = jnp.float32

    acc = bt_ref[0]
    for w in range(1, bt_ref.shape[0]):
        acc = jnp.maximum(acc, bt_ref[w])
    bigcode_ref[...] = acc

    sub_i = lax.broadcasted_iota(jnp.int32, (s, s), 0)
    lane_i = lax.broadcasted_iota(jnp.int32, (s, s), 1)
    eye = (sub_i == lane_i).astype(f32)
    dmat = eye - (sub_i == ((lane_i + 1) & (s - 1))).astype(f32)
    lane_row = lane_i[0:1, :]

    def _to_col(vrow):
        return lax.dot_general(eye, vrow, (((1,), (1,)), ((), ())),
                               preferred_element_type=f32)

    rank_ref[...] = lane_i

    def body(i, _):
        col = jnp.where(i == 0, 0, s - i)
        idxv = insT_ref[pl.ds(col, 1), :]
        idxc = _to_col(idxv.astype(f32)).astype(jnp.int32)
        eq = (rank_ref[...] == idxc).astype(jnp.int32)
        kth = jnp.sum(eq * lane_i, axis=1, keepdims=True)
        contrib = jnp.where(kth == lane_i, codes_ref[...], -1)
        win = jnp.max(contrib, axis=0, keepdims=True)
        bigrow = bigcode_ref[pl.ds(col, 1), :]
        rowcode = jnp.maximum(win, bigrow)
        val = jnp.where(rowcode >= 0, rowcode & (s - 1), 0)
        out_ref[pl.ds(i, 1), :] = val
        vf = val.astype(f32)
        vc = _to_col(vf)
        less = (vc < vf).astype(jnp.int32)
        tie = ((vc == vf) & (sub_i < lane_i)).astype(jnp.int32)
        new_rank = jnp.sum(less + tie, axis=0, keepdims=True)
        rank_ref[pl.ds(i, 1), :] = new_rank
        return 0

    lax.fori_loop(0, s, body, 0)

    def rbody(r, racc):
        outv = out_ref[pl.ds(r, 1), :]
        oh = (sub_i == outv).astype(f32)
        xrow = x_ref[pl.ds(r, 1), :]
        yrow = y_ref[pl.ds(r, 1), :]
        cd = (((1,), (0,)), ((), ()))
        sx = lax.dot_general(xrow, oh, cd, preferred_element_type=f32)
        sy = lax.dot_general(yrow, oh, cd, preferred_element_type=f32)
        dx = lax.dot_general(sx, dmat, cd, preferred_element_type=f32)
        dy = lax.dot_general(sy, dmat, cd, preferred_element_type=f32)
        nrm = jnp.sqrt(dx * dx + dy * dy)
        tot = jnp.sum(nrm)
        return racc + tot * (lane_row == r).astype(f32)

    r_ref[...] = lax.fori_loop(0, s, rbody, jnp.zeros((1, s), f32))


def _tc_sequential(tables, insT, codes, x, y):
    s = insT.shape[0]
    return pl.pallas_call(
        _tc_body,
        out_shape=(
            jax.ShapeDtypeStruct((s, s), jnp.int32),
            jax.ShapeDtypeStruct((1, s), jnp.float32),
        ),
        scratch_shapes=[
            pltpu.VMEM((s, s), jnp.int32),
            pltpu.VMEM((s, s), jnp.int32),
        ],
    )(tables, insT, codes, x, y)


def kernel(inputs, probs, actions, inserts):
    b, s = actions.shape
    last_action = actions[:, -1]
    tables = _sc_scatter_tables(inserts, last_action)
    ins_top_t = inserts[:s].T
    codes = (jnp.arange(s, dtype=jnp.int32) * s + last_action[:s]).reshape(s, 1)
    x = inputs[:s, :, 0]
    y = inputs[:s, :, 1]
    out_small, r_small = _tc_sequential(tables, ins_top_t, codes, x, y)
    actions_r = jnp.concatenate(
        [out_small, jnp.zeros((b - s, s), jnp.int32)], axis=0)
    r_full = jnp.concatenate(
        [r_small.reshape(s), jnp.zeros((b - s,), jnp.float32)], axis=0)
    return (r_full, probs, actions_r)
